# Initial kernel scaffold; baseline (speedup 1.0000x reference)
import jax
import jax.numpy as jnp
from jax.experimental import pallas as pl


def kernel(embeddings, enc_W0, enc_b0, enc_W1, enc_b1, enc_W2, enc_b2, enc_W3, enc_b3, dec_W0, dec_b0, dec_W1, dec_b1, dec_W2, dec_b2, dec_W3, dec_b3, codebooks):
    raise NotImplementedError("write your pallas kernel here")


# speedup vs baseline: 1.6232x; 1.6232x over previous
import functools

import jax
import jax.numpy as jnp
from jax.experimental import pallas as pl
from jax.experimental.pallas import tpu as pltpu

B = 16384
IN_DIM = 768
E_DIM = 32
K = 256
NC = 3
BT = 1024


_DIMS = (((1,), (0,)), ((), ()))


def _dot(a, b):
    return jax.lax.dot_general(a.astype(jnp.bfloat16), b.astype(jnp.bfloat16),
                               _DIMS, preferred_element_type=jnp.float32)


def _dot_exact(a, b):
    return jax.lax.dot_general(a, b, _DIMS, preferred_element_type=jnp.float32,
                               precision=jax.lax.Precision.HIGHEST)


def _body(x_ref,
          w0_ref, b0_ref, w1_ref, b1_ref, w2_ref, b2_ref, w3_ref, b3_ref,
          v0_ref, c0_ref, v1_ref, c1_ref, v2_ref, c2_ref, v3_ref, c3_ref,
          cb_ref, cbt_ref,
          out_ref, idx_ref, loss_ref):
    j = pl.program_id(0)

    h = x_ref[...]
    h = jnp.maximum(_dot(h, w0_ref[...]) + b0_ref[...], 0.0)
    h = jnp.maximum(_dot(h, w1_ref[...]) + b1_ref[...], 0.0)
    h = jnp.maximum(_dot(h, w2_ref[...]) + b2_ref[...], 0.0)
    z = _dot(h, w3_ref[...]) + b3_ref[...]

    iota = jax.lax.broadcasted_iota(jnp.int32, (BT, K), 1)
    r = z
    xq = jnp.zeros_like(z)
    loss = jnp.zeros((1, 1), jnp.float32)
    for i in range(NC):
        cb = cb_ref[i]
        cbt = cbt_ref[i]
        cbn = jnp.sum(cbt * cbt, axis=0, keepdims=True)
        rn = jnp.sum(r * r, axis=1, keepdims=True)
        d = rn - 2.0 * _dot(r, cbt) + cbn
        md = jnp.min(d, axis=1, keepdims=True)
        idxv = jnp.min(jnp.where(d <= md, iota, K), axis=1)
        onehot = (iota == idxv[:, None]).astype(jnp.float32)
        emb = _dot_exact(onehot, cb)
        diff = r - emb
        loss = loss + jnp.sum(diff * diff, keepdims=True)
        xq = xq + emb
        r = diff
        idx_ref[i, :] = idxv

    h = jnp.maximum(_dot(xq, v0_ref[...]) + c0_ref[...], 0.0)
    h = jnp.maximum(_dot(h, v1_ref[...]) + c1_ref[...], 0.0)
    h = jnp.maximum(_dot(h, v2_ref[...]) + c2_ref[...], 0.0)
    out_ref[...] = _dot(h, v3_ref[...]) + c3_ref[...]

    @pl.when(j == 0)
    def _():
        loss_ref[...] = jnp.zeros((1, 1), jnp.float32)

    loss_ref[...] += loss * (1.25 / (B * E_DIM))


@functools.partial(jax.jit, static_argnames=("interpret",))
def _run(embeddings,
         enc_W0, enc_b0, enc_W1, enc_b1, enc_W2, enc_b2, enc_W3, enc_b3,
         dec_W0, dec_b0, dec_W1, dec_b1, dec_W2, dec_b2, dec_W3, dec_b3,
         codebooks, interpret=False):
    nb = B // BT
    full = lambda shape: pl.BlockSpec(shape, lambda j: (0,) * len(shape))
    row2 = lambda d: pl.BlockSpec((1, d), lambda j: (0, 0))
    cbt = jnp.transpose(codebooks, (0, 2, 1))
    biases = [b.reshape(1, -1) for b in
              (enc_b0, enc_b1, enc_b2, enc_b3, dec_b0, dec_b1, dec_b2, dec_b3)]

    in_specs = [pl.BlockSpec((BT, IN_DIM), lambda j: (j, 0))]
    for w, b in zip((enc_W0, enc_W1, enc_W2, enc_W3), biases[:4]):
        in_specs += [full(w.shape), row2(b.shape[1])]
    for w, b in zip((dec_W0, dec_W1, dec_W2, dec_W3), biases[4:]):
        in_specs += [full(w.shape), row2(b.shape[1])]
    in_specs += [full((NC, K, E_DIM)), full((NC, E_DIM, K))]

    out, idx_t, loss = pl.pallas_call(
        _body,
        grid=(nb,),
        in_specs=in_specs,
        out_specs=[
            pl.BlockSpec((BT, IN_DIM), lambda j: (j, 0)),
            pl.BlockSpec((NC, BT), lambda j: (0, j)),
            pl.BlockSpec((1, 1), lambda j: (0, 0)),
        ],
        out_shape=[
            jax.ShapeDtypeStruct((B, IN_DIM), jnp.float32),
            jax.ShapeDtypeStruct((NC, B), jnp.int32),
            jax.ShapeDtypeStruct((1, 1), jnp.float32),
        ],
        compiler_params=pltpu.CompilerParams(
            dimension_semantics=("arbitrary",),
        ),
        interpret=interpret,
    )(embeddings,
      enc_W0, biases[0], enc_W1, biases[1], enc_W2, biases[2], enc_W3, biases[3],
      dec_W0, biases[4], dec_W1, biases[5], dec_W2, biases[6], dec_W3, biases[7],
      codebooks, cbt)
    return out, idx_t.T, loss[0, 0]


def kernel(embeddings,
           enc_W0, enc_b0, enc_W1, enc_b1, enc_W2, enc_b2, enc_W3, enc_b3,
           dec_W0, dec_b0, dec_W1, dec_b1, dec_W2, dec_b2, dec_W3, dec_b3,
           codebooks):
    return _run(embeddings,
                enc_W0, enc_b0, enc_W1, enc_b1, enc_W2, enc_b2, enc_W3, enc_b3,
                dec_W0, dec_b0, dec_W1, dec_b1, dec_W2, dec_b2, dec_W3, dec_b3,
                codebooks)


# speedup vs baseline: 2.1716x; 1.3379x over previous
import functools

import jax
import jax.numpy as jnp
from jax.experimental import pallas as pl
from jax.experimental.pallas import tpu as pltpu

B = 16384
IN_DIM = 768
E_DIM = 32
K = 256
NC = 3
BT = 1024


_DIMS = (((1,), (0,)), ((), ()))


def _dot(a, b):
    return jax.lax.dot_general(a.astype(jnp.bfloat16), b.astype(jnp.bfloat16),
                               _DIMS, preferred_element_type=jnp.float32)


def _split3(x):
    s0 = x.astype(jnp.bfloat16)
    r1 = x - s0.astype(jnp.float32)
    s1 = r1.astype(jnp.bfloat16)
    s2 = (r1 - s1.astype(jnp.float32)).astype(jnp.bfloat16)
    return jnp.stack([s0, s1, s2], axis=1)


def _body(x_ref,
          w0_ref, b0_ref, w1_ref, b1_ref, w2_ref, b2_ref, w3_ref, b3_ref,
          v0_ref, c0_ref, v1_ref, c1_ref, v2_ref, c2_ref, v3_ref, c3_ref,
          cbs_ref, cbt_ref,
          out_ref, idx_ref, loss_ref):
    j = pl.program_id(0)

    h = x_ref[...]
    h = jnp.maximum(_dot(h, w0_ref[...]) + b0_ref[...], 0.0)
    h = jnp.maximum(_dot(h, w1_ref[...]) + b1_ref[...], 0.0)
    h = jnp.maximum(_dot(h, w2_ref[...]) + b2_ref[...], 0.0)
    z = _dot(h, w3_ref[...]) + b3_ref[...]

    iota = jax.lax.broadcasted_iota(jnp.int32, (BT, K), 1)
    r = z
    xq = jnp.zeros_like(z)
    loss = jnp.zeros((1, 1), jnp.float32)
    for i in range(NC):
        cbt = cbt_ref[i]
        cbn = jnp.sum(cbt * cbt, axis=0, keepdims=True)
        rn = jnp.sum(r * r, axis=1, keepdims=True)
        d = rn - 2.0 * _dot(r, cbt) + cbn
        md = jnp.min(d, axis=1, keepdims=True)
        idxv = jnp.min(jnp.where(d <= md, iota, K), axis=1)
        onehot = (iota == idxv[:, None]).astype(jnp.bfloat16)
        emb = ((_dot(onehot, cbs_ref[i, 0]) + _dot(onehot, cbs_ref[i, 1]))
               + _dot(onehot, cbs_ref[i, 2]))
        diff = r - emb
        loss = loss + jnp.sum(diff * diff, keepdims=True)
        xq = xq + emb
        r = diff
        idx_ref[i, :] = idxv

    h = jnp.maximum(_dot(xq, v0_ref[...]) + c0_ref[...], 0.0)
    h = jnp.maximum(_dot(h, v1_ref[...]) + c1_ref[...], 0.0)
    h = jnp.maximum(_dot(h, v2_ref[...]) + c2_ref[...], 0.0)
    out_ref[...] = _dot(h, v3_ref[...]) + c3_ref[...]

    @pl.when(j == 0)
    def _():
        loss_ref[...] = jnp.zeros((1, 1), jnp.float32)

    loss_ref[...] += loss * (1.25 / (B * E_DIM))


@functools.partial(jax.jit, static_argnames=("interpret",))
def _run(embeddings,
         enc_W0, enc_b0, enc_W1, enc_b1, enc_W2, enc_b2, enc_W3, enc_b3,
         dec_W0, dec_b0, dec_W1, dec_b1, dec_W2, dec_b2, dec_W3, dec_b3,
         codebooks, interpret=False):
    nb = B // BT
    full = lambda shape: pl.BlockSpec(shape, lambda j: (0,) * len(shape))
    row2 = lambda d: pl.BlockSpec((1, d), lambda j: (0, 0))
    cbt = jnp.transpose(codebooks, (0, 2, 1))
    biases = [b.reshape(1, -1) for b in
              (enc_b0, enc_b1, enc_b2, enc_b3, dec_b0, dec_b1, dec_b2, dec_b3)]

    in_specs = [pl.BlockSpec((BT, IN_DIM), lambda j: (j, 0))]
    for w, b in zip((enc_W0, enc_W1, enc_W2, enc_W3), biases[:4]):
        in_specs += [full(w.shape), row2(b.shape[1])]
    for w, b in zip((dec_W0, dec_W1, dec_W2, dec_W3), biases[4:]):
        in_specs += [full(w.shape), row2(b.shape[1])]
    in_specs += [full((NC, 3, K, E_DIM)), full((NC, E_DIM, K))]

    out, idx_t, loss = pl.pallas_call(
        _body,
        grid=(nb,),
        in_specs=in_specs,
        out_specs=[
            pl.BlockSpec((BT, IN_DIM), lambda j: (j, 0)),
            pl.BlockSpec((NC, BT), lambda j: (0, j)),
            pl.BlockSpec((1, 1), lambda j: (0, 0)),
        ],
        out_shape=[
            jax.ShapeDtypeStruct((B, IN_DIM), jnp.float32),
            jax.ShapeDtypeStruct((NC, B), jnp.int32),
            jax.ShapeDtypeStruct((1, 1), jnp.float32),
        ],
        compiler_params=pltpu.CompilerParams(
            dimension_semantics=("arbitrary",),
        ),
        interpret=interpret,
    )(embeddings,
      enc_W0, biases[0], enc_W1, biases[1], enc_W2, biases[2], enc_W3, biases[3],
      dec_W0, biases[4], dec_W1, biases[5], dec_W2, biases[6], dec_W3, biases[7],
      _split3(codebooks), cbt)
    return out, idx_t.T, loss[0, 0]


def kernel(embeddings,
           enc_W0, enc_b0, enc_W1, enc_b1, enc_W2, enc_b2, enc_W3, enc_b3,
           dec_W0, dec_b0, dec_W1, dec_b1, dec_W2, dec_b2, dec_W3, dec_b3,
           codebooks):
    return _run(embeddings,
                enc_W0, enc_b0, enc_W1, enc_b1, enc_W2, enc_b2, enc_W3, enc_b3,
                dec_W0, dec_b0, dec_W1, dec_b1, dec_W2, dec_b2, dec_W3, dec_b3,
                codebooks)
